# baseline (speedup 1.0000x reference)
import functools

import jax
import jax.numpy as jnp
from jax import lax
from jax.experimental import pallas as pl
from jax.experimental.pallas import tpu as pltpu
from jax.experimental.pallas import tpu_sc as plsc

N = 10000
E = 320000
G = 64

_N_BLK = 1000
_E_BLK = 2000


def _proj_body(h_ref, wa_ref, ba_ref, a_ref, b_ref, *, fin):
    h = h_ref[...]
    wa_top = wa_ref[:fin, :]
    wa_bot = wa_ref[fin:, :]
    a_ref[...] = jnp.dot(h, wa_top - wa_bot, preferred_element_type=jnp.float32, precision=lax.Precision.HIGHEST) + ba_ref[0, :]
    b_ref[...] = jnp.dot(h, wa_bot, preferred_element_type=jnp.float32, precision=lax.Precision.HIGHEST)


def _proj(h, wa, ba):
    fin = h.shape[1]
    hc = wa.shape[1]
    grid = N // _N_BLK
    return pl.pallas_call(
        functools.partial(_proj_body, fin=fin),
        grid=(grid,),
        in_specs=[
            pl.BlockSpec((_N_BLK, fin), lambda i: (i, 0)),
            pl.BlockSpec((2 * fin, hc), lambda i: (0, 0)),
            pl.BlockSpec((1, hc), lambda i: (0, 0)),
        ],
        out_specs=[
            pl.BlockSpec((_N_BLK, hc), lambda i: (i, 0)),
            pl.BlockSpec((_N_BLK, hc), lambda i: (i, 0)),
        ],
        out_shape=[
            jax.ShapeDtypeStruct((N, hc), jnp.float32),
            jax.ShapeDtypeStruct((N, hc), jnp.float32),
        ],
    )(h, wa, ba.reshape(1, hc))


def _edge_mlp_body(pre_ref, wb_ref, bb_ref, m_ref):
    pre = jnp.maximum(pre_ref[...], 0.0)
    m_ref[...] = jnp.dot(pre, wb_ref[...], preferred_element_type=jnp.float32, precision=lax.Precision.HIGHEST) + bb_ref[0, :]


def _edge_mlp(pre, wb, bb):
    hc_in, hc = wb.shape
    grid = E // _E_BLK
    return pl.pallas_call(
        _edge_mlp_body,
        grid=(grid,),
        in_specs=[
            pl.BlockSpec((_E_BLK, hc_in), lambda i: (i, 0)),
            pl.BlockSpec((hc_in, hc), lambda i: (0, 0)),
            pl.BlockSpec((1, hc), lambda i: (0, 0)),
        ],
        out_specs=pl.BlockSpec((_E_BLK, hc), lambda i: (i, 0)),
        out_shape=jax.ShapeDtypeStruct((E, hc), jnp.float32),
    )(pre, wb, bb.reshape(1, hc))


def _pool_head_body(h_ref, bi_ref, g_ref, be_ref, mu_ref, va_ref,
                    wl1_ref, bl1_ref, wl2_ref, bl2_ref, wl3_ref, bl3_ref,
                    wh_ref, bh_ref, out_ref):
    h = h_ref[...]
    h = jnp.where(h == -jnp.inf, 0.0, h)
    h = (h - mu_ref[0, :]) / jnp.sqrt(va_ref[0, :] + 1e-5) * g_ref[0, :] + be_ref[0, :]
    h = jnp.maximum(h, 0.0)
    bi = bi_ref[0, :]
    oh = (bi[None, :] == jax.lax.broadcasted_iota(jnp.int32, (G, N), 0)).astype(jnp.float32)
    sums = jnp.dot(oh, h, preferred_element_type=jnp.float32, precision=lax.Precision.HIGHEST)
    cnt = jnp.sum(oh, axis=1)
    p = sums / jnp.maximum(cnt, 1.0)[:, None]
    p = jnp.maximum(jnp.dot(p, wl1_ref[...], preferred_element_type=jnp.float32, precision=lax.Precision.HIGHEST) + bl1_ref[0, :], 0.0)
    p = jnp.maximum(jnp.dot(p, wl2_ref[...], preferred_element_type=jnp.float32, precision=lax.Precision.HIGHEST) + bl2_ref[0, :], 0.0)
    p = jnp.maximum(jnp.dot(p, wl3_ref[...], preferred_element_type=jnp.float32, precision=lax.Precision.HIGHEST) + bl3_ref[0, :], 0.0)
    out_ref[...] = jnp.dot(p, wh_ref[...], preferred_element_type=jnp.float32, precision=lax.Precision.HIGHEST) + bh_ref[0, :]


def _pool_head(h, batch_idx, bn_gamma, bn_beta, bn_mean, bn_var,
               wl1, bl1, wl2, bl2, wl3, bl3, wh, bh):
    hc = h.shape[1]
    full = lambda a: pl.BlockSpec(a.shape, lambda: tuple(0 for _ in a.shape))
    args = [h, batch_idx.reshape(1, N).astype(jnp.int32),
            bn_gamma.reshape(1, hc), bn_beta.reshape(1, hc),
            bn_mean.reshape(1, hc), bn_var.reshape(1, hc),
            wl1, bl1.reshape(1, -1), wl2, bl2.reshape(1, -1),
            wl3, bl3.reshape(1, -1), wh, bh.reshape(1, -1)]
    return pl.pallas_call(
        _pool_head_body,
        in_specs=[full(a) for a in args],
        out_specs=pl.BlockSpec((G, wh.shape[1]), lambda: (0, 0)),
        out_shape=jax.ShapeDtypeStruct((G, wh.shape[1]), jnp.float32),
    )(*args)


def _sc_gather_add(a, b, dst, src):
    hc = a.shape[1]
    info = plsc.get_sparse_core_info()
    nw = info.num_cores * info.num_subcores
    epw = E // nw
    chunk = 1000
    nchunks = epw // chunk
    mesh = plsc.VectorSubcoreMesh(core_axis_name="c", subcore_axis_name="s")

    @functools.partial(
        pl.kernel, mesh=mesh,
        compiler_params=pltpu.CompilerParams(use_tc_tiling_on_sc=False),
        out_type=jax.ShapeDtypeStruct((E, hc), jnp.float32),
        scratch_types=[
            pltpu.VMEM((chunk,), jnp.int32),
            pltpu.VMEM((chunk,), jnp.int32),
            pltpu.VMEM((chunk, hc), jnp.float32),
            pltpu.VMEM((chunk, hc), jnp.float32),
            pltpu.SemaphoreType.DMA,
            pltpu.SemaphoreType.DMA,
        ],
    )
    def k(a_hbm, b_hbm, dst_hbm, src_hbm, pre_hbm, dstv, srcv, rows_a, rows_b,
          sem_a, sem_b):
        wid = lax.axis_index("s") * info.num_cores + lax.axis_index("c")
        base = wid * epw

        def chunk_body(c, carry):
            e0 = base + c * chunk
            pltpu.sync_copy(dst_hbm.at[pl.ds(e0, chunk)], dstv)
            pltpu.sync_copy(src_hbm.at[pl.ds(e0, chunk)], srcv)
            cp_a = pltpu.async_copy(a_hbm.at[dstv], rows_a, sem_a)
            cp_b = pltpu.async_copy(b_hbm.at[srcv], rows_b, sem_b)
            cp_a.wait()
            cp_b.wait()

            def add_body(r, c2):
                for f in range(hc // 16):
                    sl = pl.ds(f * 16, 16)
                    rows_a[r, sl] = rows_a[r, sl] + rows_b[r, sl]
                return c2

            lax.fori_loop(0, chunk, add_body, 0)
            pltpu.sync_copy(rows_a, pre_hbm.at[pl.ds(e0, chunk)])
            return carry

        lax.fori_loop(0, nchunks, chunk_body, 0)

    return k(a, b, dst, src)


def kernel(x, edge_index, batch_idx, w1a, b1a, w1b, b1b, w2a, b2a, w2b, b2b,
           w3a, b3a, w3b, b3b, bn_gamma, bn_beta, bn_mean, bn_var,
           wl1, bl1, wl2, bl2, wl3, bl3, wh, bh):
    src, dst = edge_index[0], edge_index[1]
    h = x
    for i, (wa, ba, wb, bb) in enumerate(
            [(w1a, b1a, w1b, b1b), (w2a, b2a, w2b, b2b), (w3a, b3a, w3b, b3b)]):
        a, b = _proj(h, wa, ba)
        pre = _sc_gather_add(a, b, dst, src)
        m = _edge_mlp(pre, wb, bb)
        seg = jax.ops.segment_max(m, dst, num_segments=N)
        if i < 2:
            h = jnp.maximum(seg, 0.0)
        else:
            h = seg
    return _pool_head(h, batch_idx, bn_gamma, bn_beta, bn_mean, bn_var,
                      wl1, bl1, wl2, bl2, wl3, bl3, wh, bh)


# speedup vs baseline: 1.0477x; 1.0477x over previous
import functools

import jax
import jax.numpy as jnp
from jax import lax
from jax.experimental import pallas as pl
from jax.experimental.pallas import tpu as pltpu
from jax.experimental.pallas import tpu_sc as plsc

N = 10000
E = 320000
G = 64

_N_BLK = 1000
_E_BLK = 2000


def _proj_body(h_ref, wa_ref, ba_ref, a_ref, b_ref, *, fin):
    h = h_ref[...]
    wa_top = wa_ref[:fin, :]
    wa_bot = wa_ref[fin:, :]
    a_ref[...] = jnp.dot(h, wa_top - wa_bot, preferred_element_type=jnp.float32) + ba_ref[0, :]
    b_ref[...] = jnp.dot(h, wa_bot, preferred_element_type=jnp.float32)


def _proj(h, wa, ba):
    fin = h.shape[1]
    hc = wa.shape[1]
    grid = N // _N_BLK
    return pl.pallas_call(
        functools.partial(_proj_body, fin=fin),
        grid=(grid,),
        in_specs=[
            pl.BlockSpec((_N_BLK, fin), lambda i: (i, 0)),
            pl.BlockSpec((2 * fin, hc), lambda i: (0, 0)),
            pl.BlockSpec((1, hc), lambda i: (0, 0)),
        ],
        out_specs=[
            pl.BlockSpec((_N_BLK, hc), lambda i: (i, 0)),
            pl.BlockSpec((_N_BLK, hc), lambda i: (i, 0)),
        ],
        out_shape=[
            jax.ShapeDtypeStruct((N, hc), jnp.float32),
            jax.ShapeDtypeStruct((N, hc), jnp.float32),
        ],
    )(h, wa, ba.reshape(1, hc))


def _edge_mlp_body(pre_ref, wb_ref, bb_ref, m_ref):
    pre = jnp.maximum(pre_ref[...], 0.0)
    m_ref[...] = jnp.dot(pre, wb_ref[...], preferred_element_type=jnp.float32) + bb_ref[0, :]


def _edge_mlp(pre, wb, bb):
    hc_in, hc = wb.shape
    grid = E // _E_BLK
    return pl.pallas_call(
        _edge_mlp_body,
        grid=(grid,),
        in_specs=[
            pl.BlockSpec((_E_BLK, hc_in), lambda i: (i, 0)),
            pl.BlockSpec((hc_in, hc), lambda i: (0, 0)),
            pl.BlockSpec((1, hc), lambda i: (0, 0)),
        ],
        out_specs=pl.BlockSpec((_E_BLK, hc), lambda i: (i, 0)),
        out_shape=jax.ShapeDtypeStruct((E, hc), jnp.float32),
    )(pre, wb, bb.reshape(1, hc))


def _pool_head_body(h_ref, bi_ref, g_ref, be_ref, mu_ref, va_ref,
                    wl1_ref, bl1_ref, wl2_ref, bl2_ref, wl3_ref, bl3_ref,
                    wh_ref, bh_ref, out_ref):
    h = h_ref[...]
    h = jnp.where(h == -jnp.inf, 0.0, h)
    h = (h - mu_ref[0, :]) / jnp.sqrt(va_ref[0, :] + 1e-5) * g_ref[0, :] + be_ref[0, :]
    h = jnp.maximum(h, 0.0)
    bi = bi_ref[0, :]
    oh = (bi[None, :] == jax.lax.broadcasted_iota(jnp.int32, (G, N), 0)).astype(jnp.float32)
    sums = jnp.dot(oh, h, preferred_element_type=jnp.float32)
    cnt = jnp.sum(oh, axis=1)
    p = sums / jnp.maximum(cnt, 1.0)[:, None]
    p = jnp.maximum(jnp.dot(p, wl1_ref[...], preferred_element_type=jnp.float32) + bl1_ref[0, :], 0.0)
    p = jnp.maximum(jnp.dot(p, wl2_ref[...], preferred_element_type=jnp.float32) + bl2_ref[0, :], 0.0)
    p = jnp.maximum(jnp.dot(p, wl3_ref[...], preferred_element_type=jnp.float32) + bl3_ref[0, :], 0.0)
    out_ref[...] = jnp.dot(p, wh_ref[...], preferred_element_type=jnp.float32) + bh_ref[0, :]


def _pool_head(h, batch_idx, bn_gamma, bn_beta, bn_mean, bn_var,
               wl1, bl1, wl2, bl2, wl3, bl3, wh, bh):
    hc = h.shape[1]
    full = lambda a: pl.BlockSpec(a.shape, lambda: tuple(0 for _ in a.shape))
    args = [h, batch_idx.reshape(1, N).astype(jnp.int32),
            bn_gamma.reshape(1, hc), bn_beta.reshape(1, hc),
            bn_mean.reshape(1, hc), bn_var.reshape(1, hc),
            wl1, bl1.reshape(1, -1), wl2, bl2.reshape(1, -1),
            wl3, bl3.reshape(1, -1), wh, bh.reshape(1, -1)]
    return pl.pallas_call(
        _pool_head_body,
        in_specs=[full(a) for a in args],
        out_specs=pl.BlockSpec((G, wh.shape[1]), lambda: (0, 0)),
        out_shape=jax.ShapeDtypeStruct((G, wh.shape[1]), jnp.float32),
    )(*args)


def _sc_gather_add(a, b, dst, src):
    hc = a.shape[1]
    info = plsc.get_sparse_core_info()
    nw = info.num_cores * info.num_subcores
    epw = E // nw
    chunk = 1000
    nchunks = epw // chunk
    mesh = plsc.VectorSubcoreMesh(core_axis_name="c", subcore_axis_name="s")

    @functools.partial(
        pl.kernel, mesh=mesh,
        compiler_params=pltpu.CompilerParams(use_tc_tiling_on_sc=False),
        out_type=jax.ShapeDtypeStruct((E, hc), jnp.float32),
        scratch_types=[
            pltpu.VMEM((chunk,), jnp.int32),
            pltpu.VMEM((chunk,), jnp.int32),
            pltpu.VMEM((chunk, hc), jnp.float32),
            pltpu.VMEM((chunk, hc), jnp.float32),
            pltpu.SemaphoreType.DMA,
            pltpu.SemaphoreType.DMA,
        ],
    )
    def k(a_hbm, b_hbm, dst_hbm, src_hbm, pre_hbm, dstv, srcv, rows_a, rows_b,
          sem_a, sem_b):
        wid = lax.axis_index("s") * info.num_cores + lax.axis_index("c")
        base = wid * epw

        def chunk_body(c, carry):
            e0 = base + c * chunk
            pltpu.sync_copy(dst_hbm.at[pl.ds(e0, chunk)], dstv)
            pltpu.sync_copy(src_hbm.at[pl.ds(e0, chunk)], srcv)
            cp_a = pltpu.async_copy(a_hbm.at[dstv], rows_a, sem_a)
            cp_b = pltpu.async_copy(b_hbm.at[srcv], rows_b, sem_b)
            cp_a.wait()
            cp_b.wait()

            def add_body(r, c2):
                for f in range(hc // 16):
                    sl = pl.ds(f * 16, 16)
                    rows_a[r, sl] = rows_a[r, sl] + rows_b[r, sl]
                return c2

            lax.fori_loop(0, chunk, add_body, 0)
            pltpu.sync_copy(rows_a, pre_hbm.at[pl.ds(e0, chunk)])
            return carry

        lax.fori_loop(0, nchunks, chunk_body, 0)

    return k(a, b, dst, src)


def kernel(x, edge_index, batch_idx, w1a, b1a, w1b, b1b, w2a, b2a, w2b, b2b,
           w3a, b3a, w3b, b3b, bn_gamma, bn_beta, bn_mean, bn_var,
           wl1, bl1, wl2, bl2, wl3, bl3, wh, bh):
    src, dst = edge_index[0], edge_index[1]
    h = x
    for i, (wa, ba, wb, bb) in enumerate(
            [(w1a, b1a, w1b, b1b), (w2a, b2a, w2b, b2b), (w3a, b3a, w3b, b3b)]):
        a, b = _proj(h, wa, ba)
        pre = _sc_gather_add(a, b, dst, src)
        m = _edge_mlp(pre, wb, bb)
        seg = jax.ops.segment_max(m, dst, num_segments=N)
        if i < 2:
            h = jnp.maximum(seg, 0.0)
        else:
            h = seg
    return _pool_head(h, batch_idx, bn_gamma, bn_beta, bn_mean, bn_var,
                      wl1, bl1, wl2, bl2, wl3, bl3, wh, bh)


# speedup vs baseline: 1.2296x; 1.1737x over previous
import functools

import jax
import jax.numpy as jnp
from jax import lax
from jax.experimental import pallas as pl
from jax.experimental.pallas import tpu as pltpu
from jax.experimental.pallas import tpu_sc as plsc

N = 10000
E = 320000
G = 64

_N_BLK = 1000
_E_BLK = 2000


def _proj_body(h_ref, wa_ref, ba_ref, a_ref, b_ref, *, fin):
    h = h_ref[...]
    wa_top = wa_ref[:fin, :]
    wa_bot = wa_ref[fin:, :]
    a_ref[...] = jnp.dot(h, wa_top - wa_bot, preferred_element_type=jnp.float32) + ba_ref[0, :]
    b_ref[...] = jnp.dot(h, wa_bot, preferred_element_type=jnp.float32)


def _proj(h, wa, ba):
    fin = h.shape[1]
    hc = wa.shape[1]
    grid = N // _N_BLK
    return pl.pallas_call(
        functools.partial(_proj_body, fin=fin),
        grid=(grid,),
        in_specs=[
            pl.BlockSpec((_N_BLK, fin), lambda i: (i, 0)),
            pl.BlockSpec((2 * fin, hc), lambda i: (0, 0)),
            pl.BlockSpec((1, hc), lambda i: (0, 0)),
        ],
        out_specs=[
            pl.BlockSpec((_N_BLK, hc), lambda i: (i, 0)),
            pl.BlockSpec((_N_BLK, hc), lambda i: (i, 0)),
        ],
        out_shape=[
            jax.ShapeDtypeStruct((N, hc), jnp.float32),
            jax.ShapeDtypeStruct((N, hc), jnp.float32),
        ],
    )(h, wa, ba.reshape(1, hc))


def _edge_mlp_body(pre_ref, wb_ref, bb_ref, m_ref):
    pre = jnp.maximum(pre_ref[...], 0.0)
    m_ref[...] = jnp.dot(pre, wb_ref[...], preferred_element_type=jnp.float32) + bb_ref[0, :]


def _edge_mlp(pre, wb, bb):
    hc_in, hc = wb.shape
    ne = pre.shape[0]
    grid = ne // _E_BLK
    return pl.pallas_call(
        _edge_mlp_body,
        grid=(grid,),
        in_specs=[
            pl.BlockSpec((_E_BLK, hc_in), lambda i: (i, 0)),
            pl.BlockSpec((hc_in, hc), lambda i: (0, 0)),
            pl.BlockSpec((1, hc), lambda i: (0, 0)),
        ],
        out_specs=pl.BlockSpec((_E_BLK, hc), lambda i: (i, 0)),
        out_shape=jax.ShapeDtypeStruct((ne, hc), jnp.float32),
    )(pre, wb, bb.reshape(1, hc))


def _pool_head_body(h_ref, bi_ref, g_ref, be_ref, mu_ref, va_ref,
                    wl1_ref, bl1_ref, wl2_ref, bl2_ref, wl3_ref, bl3_ref,
                    wh_ref, bh_ref, out_ref):
    h = h_ref[...]
    h = jnp.where(h == -jnp.inf, 0.0, h)
    h = (h - mu_ref[0, :]) / jnp.sqrt(va_ref[0, :] + 1e-5) * g_ref[0, :] + be_ref[0, :]
    h = jnp.maximum(h, 0.0)
    bi = bi_ref[0, :]
    oh = (bi[None, :] == jax.lax.broadcasted_iota(jnp.int32, (G, N), 0)).astype(jnp.float32)
    sums = jnp.dot(oh, h, preferred_element_type=jnp.float32)
    cnt = jnp.sum(oh, axis=1)
    p = sums / jnp.maximum(cnt, 1.0)[:, None]
    p = jnp.maximum(jnp.dot(p, wl1_ref[...], preferred_element_type=jnp.float32) + bl1_ref[0, :], 0.0)
    p = jnp.maximum(jnp.dot(p, wl2_ref[...], preferred_element_type=jnp.float32) + bl2_ref[0, :], 0.0)
    p = jnp.maximum(jnp.dot(p, wl3_ref[...], preferred_element_type=jnp.float32) + bl3_ref[0, :], 0.0)
    out_ref[...] = jnp.dot(p, wh_ref[...], preferred_element_type=jnp.float32) + bh_ref[0, :]


def _pool_head(h, batch_idx, bn_gamma, bn_beta, bn_mean, bn_var,
               wl1, bl1, wl2, bl2, wl3, bl3, wh, bh):
    hc = h.shape[1]
    full = lambda a: pl.BlockSpec(a.shape, lambda: tuple(0 for _ in a.shape))
    args = [h, batch_idx.reshape(1, N).astype(jnp.int32),
            bn_gamma.reshape(1, hc), bn_beta.reshape(1, hc),
            bn_mean.reshape(1, hc), bn_var.reshape(1, hc),
            wl1, bl1.reshape(1, -1), wl2, bl2.reshape(1, -1),
            wl3, bl3.reshape(1, -1), wh, bh.reshape(1, -1)]
    return pl.pallas_call(
        _pool_head_body,
        in_specs=[full(a) for a in args],
        out_specs=pl.BlockSpec((G, wh.shape[1]), lambda: (0, 0)),
        out_shape=jax.ShapeDtypeStruct((G, wh.shape[1]), jnp.float32),
    )(*args)


def _sc_gather_add(a, b, dst, src, ne):
    hc = a.shape[1]
    info = plsc.get_sparse_core_info()
    nw = info.num_cores * info.num_subcores
    epw = ne // nw
    chunk = 1000
    nchunks = epw // chunk
    mesh = plsc.VectorSubcoreMesh(core_axis_name="c", subcore_axis_name="s")

    @functools.partial(
        pl.kernel, mesh=mesh,
        compiler_params=pltpu.CompilerParams(use_tc_tiling_on_sc=False),
        out_type=jax.ShapeDtypeStruct((ne, hc), jnp.float32),
        scratch_types=[
            pltpu.VMEM((chunk,), jnp.int32),
            pltpu.VMEM((chunk,), jnp.int32),
            pltpu.VMEM((chunk, hc), jnp.float32),
            pltpu.VMEM((chunk, hc), jnp.float32),
            pltpu.SemaphoreType.DMA,
            pltpu.SemaphoreType.DMA,
        ],
    )
    def k(a_hbm, b_hbm, dst_hbm, src_hbm, pre_hbm, dstv, srcv, rows_a, rows_b,
          sem_a, sem_b):
        wid = lax.axis_index("s") * info.num_cores + lax.axis_index("c")
        base = wid * epw

        def chunk_body(c, carry):
            e0 = base + c * chunk
            pltpu.sync_copy(dst_hbm.at[pl.ds(e0, chunk)], dstv)
            pltpu.sync_copy(src_hbm.at[pl.ds(e0, chunk)], srcv)
            cp_a = pltpu.async_copy(a_hbm.at[dstv], rows_a, sem_a)
            cp_b = pltpu.async_copy(b_hbm.at[srcv], rows_b, sem_b)
            cp_a.wait()
            cp_b.wait()

            def add_body(r, c2):
                for f in range(hc // 16):
                    sl = pl.ds(f * 16, 16)
                    rows_a[r, sl] = rows_a[r, sl] + rows_b[r, sl]
                return c2

            lax.fori_loop(0, chunk, add_body, 0)
            pltpu.sync_copy(rows_a, pre_hbm.at[pl.ds(e0, chunk)])
            return carry

        lax.fori_loop(0, nchunks, chunk_body, 0)

    return k(a, b, dst, src)


def kernel(x, edge_index, batch_idx, w1a, b1a, w1b, b1b, w2a, b2a, w2b, b2b,
           w3a, b3a, w3b, b3b, bn_gamma, bn_beta, bn_mean, bn_var,
           wl1, bl1, wl2, bl2, wl3, bl3, wh, bh):
    src, dst = edge_index[0], edge_index[1]
    e2 = E // 2
    halves = [(dst[:e2], src[:e2]), (dst[e2:], src[e2:])]
    h = x
    for i, (wa, ba, wb, bb) in enumerate(
            [(w1a, b1a, w1b, b1b), (w2a, b2a, w2b, b2b), (w3a, b3a, w3b, b3b)]):
        a, b = _proj(h, wa, ba)
        segs = []
        for dh, sh in halves:
            pre = _sc_gather_add(a, b, dh, sh, e2)
            m = _edge_mlp(pre, wb, bb)
            segs.append(jax.ops.segment_max(m, dh, num_segments=N))
        seg = jnp.maximum(segs[0], segs[1])
        if i < 2:
            h = jnp.maximum(seg, 0.0)
        else:
            h = seg
    return _pool_head(h, batch_idx, bn_gamma, bn_beta, bn_mean, bn_var,
                      wl1, bl1, wl2, bl2, wl3, bl3, wh, bh)
